# baseline (speedup 1.0000x reference)
import functools

import jax
import jax.numpy as jnp
from jax import lax
from jax.experimental import pallas as pl
from jax.experimental.pallas import tpu as pltpu
from jax.experimental.pallas import tpu_sc as plsc

BATCH = 2
SEQ = 2048
D = 768
N = BATCH * SEQ
NW = 32
N_PER_W = N // NW
W_PER_B = NW // BATCH
CHUNK = 16
N_CHUNKS = N_PER_W // CHUNK
NB = 5
LOOKAHEAD = 4
LANES = 16


def _sc_embed_add(ids, hidden, table):
    mesh = plsc.VectorSubcoreMesh(core_axis_name="c", subcore_axis_name="s")

    scratch = [pltpu.VMEM((N_PER_W,), jnp.int32)]
    scratch += [pltpu.VMEM((CHUNK, D), jnp.float32) for _ in range(2 * NB)]
    scratch += [pltpu.SemaphoreType.DMA for _ in range(3 * NB)]

    @functools.partial(
        pl.kernel,
        mesh=mesh,
        out_type=jax.ShapeDtypeStruct((BATCH, SEQ, D), jnp.float32),
        scratch_types=scratch,
    )
    def k(ids_hbm, hid_hbm, table_hbm, out_hbm, idx_v, *bufs):
        rows = bufs[0:NB]
        hid = bufs[NB:2 * NB]
        gsem = bufs[2 * NB:3 * NB]
        hsem = bufs[3 * NB:4 * NB]
        osem = bufs[4 * NB:5 * NB]

        wid = lax.axis_index("s") * 2 + lax.axis_index("c")
        bi = wid // W_PER_B
        seq0 = (wid % W_PER_B) * N_PER_W

        g = [None] * N_CHUNKS
        h = [None] * N_CHUNKS
        o = [None] * N_CHUNKS

        def issue_g(c):
            b = c % NB
            g[c] = pltpu.async_copy(
                table_hbm.at[idx_v.at[pl.ds(c * CHUNK, CHUNK)]], rows[b], gsem[b]
            )

        def issue_h(c):
            b = c % NB
            h[c] = pltpu.async_copy(
                hid_hbm.at[bi, pl.ds(seq0 + c * CHUNK, CHUNK)], hid[b], hsem[b]
            )

        def issue(c):
            issue_g(c)
            issue_h(c)

        for c in range(LOOKAHEAD):
            issue_h(c)
        pltpu.sync_copy(ids_hbm.at[bi, pl.ds(seq0, N_PER_W)], idx_v)
        for c in range(LOOKAHEAD):
            issue_g(c)
        for c in range(N_CHUNKS):
            b = c % NB
            if c + LOOKAHEAD < N_CHUNKS:
                if c + LOOKAHEAD - NB >= 0:
                    o[c + LOOKAHEAD - NB].wait()
                issue(c + LOOKAHEAD)
            g[c].wait()
            h[c].wait()

            def add_row(i, carry):
                for j in range(D // LANES):
                    sl = pl.ds(j * LANES, LANES)
                    rows[b][i, sl] = rows[b][i, sl] + hid[b][i, sl]
                return carry

            lax.fori_loop(0, CHUNK, add_row, 0)
            o[c] = pltpu.async_copy(
                rows[b], out_hbm.at[bi, pl.ds(seq0 + c * CHUNK, CHUNK)], osem[b]
            )
        for c in range(max(0, N_CHUNKS - NB), N_CHUNKS):
            o[c].wait()

    return k(ids, hidden, table)


def kernel(input_ids, hidden_states, positions, embed_table):
    return _sc_embed_add(
        input_ids.astype(jnp.int32), hidden_states, embed_table
    )


# speedup vs baseline: 1.0863x; 1.0863x over previous
import functools

import jax
import jax.numpy as jnp
from jax import lax
from jax.experimental import pallas as pl
from jax.experimental.pallas import tpu as pltpu
from jax.experimental.pallas import tpu_sc as plsc

BATCH = 2
SEQ = 2048
D = 768
N = BATCH * SEQ
NW = 32
N_PER_W = N // NW
W_PER_B = NW // BATCH
CHUNK = 16
N_CHUNKS = N_PER_W // CHUNK
NB = 4
LOOKAHEAD = 3
GROUPS = N_CHUNKS // NB
LANES = 16


def _sc_embed_add(ids, hidden, table):
    mesh = plsc.VectorSubcoreMesh(core_axis_name="c", subcore_axis_name="s")

    scratch = [pltpu.VMEM((N_PER_W,), jnp.int32)]
    scratch += [pltpu.VMEM((CHUNK, D), jnp.float32) for _ in range(2 * NB)]
    scratch += [pltpu.SemaphoreType.DMA for _ in range(3 * NB)]

    @functools.partial(
        pl.kernel,
        mesh=mesh,
        out_type=jax.ShapeDtypeStruct((BATCH, SEQ, D), jnp.float32),
        scratch_types=scratch,
    )
    def k(ids_hbm, hid_hbm, table_hbm, out_hbm, idx_v, *bufs):
        rows = bufs[0:NB]
        hid = bufs[NB:2 * NB]
        gsem = bufs[2 * NB:3 * NB]
        hsem = bufs[3 * NB:4 * NB]
        osem = bufs[4 * NB:5 * NB]

        wid = lax.axis_index("s") * 2 + lax.axis_index("c")
        bi = wid // W_PER_B
        seq0 = (wid % W_PER_B) * N_PER_W
        pltpu.sync_copy(ids_hbm.at[bi, pl.ds(seq0, N_PER_W)], idx_v)

        g = [None] * N_CHUNKS
        h = [None] * N_CHUNKS
        o = [None] * N_CHUNKS

        def issue(c):
            b = c % NB
            g[c] = pltpu.async_copy(
                table_hbm.at[idx_v.at[pl.ds(c * CHUNK, CHUNK)]], rows[b], gsem[b]
            )
            h[c] = pltpu.async_copy(
                hid_hbm.at[bi, pl.ds(seq0 + c * CHUNK, CHUNK)], hid[b], hsem[b]
            )

        def mk_g(c, b):
            return pltpu.make_async_copy(
                table_hbm.at[idx_v.at[pl.ds(c * CHUNK, CHUNK)]], rows[b], gsem[b]
            )

        def mk_h(c, b):
            return pltpu.make_async_copy(
                hid_hbm.at[bi, pl.ds(seq0 + c * CHUNK, CHUNK)], hid[b], hsem[b]
            )

        def mk_o(c, b):
            return pltpu.make_async_copy(
                rows[b], out_hbm.at[bi, pl.ds(seq0 + c * CHUNK, CHUNK)], osem[b]
            )

        for c in range(LOOKAHEAD):
            issue(c)

        def group(gi, carry):
            for b in range(NB):
                c = gi * NB + b
                bn = (b + LOOKAHEAD) % NB
                bp = (b - 1) % NB
                if b == 0:
                    @pl.when(gi > 0)
                    def _():
                        mk_o(c - 1, bp).wait()
                    mk_g(c + LOOKAHEAD, bn).start()
                    mk_h(c + LOOKAHEAD, bn).start()
                else:
                    @pl.when(gi == 0)
                    def _():
                        mk_o(c - 1, bp).wait()
                        mk_g(c + LOOKAHEAD, bn).start()
                        mk_h(c + LOOKAHEAD, bn).start()
                mk_g(c, b).wait()
                mk_h(c, b).wait()

                def add_row(i, carry2):
                    for j in range(D // LANES):
                        sl = pl.ds(j * LANES, LANES)
                        rows[b][i, sl] = rows[b][i, sl] + hid[b][i, sl]
                    return carry2

                lax.fori_loop(0, CHUNK, add_row, 0)
                mk_o(c, b).start()
            return carry

        lax.fori_loop(0, GROUPS, group, 0)
        for c in range(N_CHUNKS - NB, N_CHUNKS):
            mk_o(c, c % NB).wait()

    return k(ids, hidden, table)


def kernel(input_ids, hidden_states, positions, embed_table):
    return _sc_embed_add(
        input_ids.astype(jnp.int32), hidden_states, embed_table
    )
